# baseline (speedup 1.0000x reference)
import functools

import jax
import jax.numpy as jnp
from jax import lax
from jax.experimental import pallas as pl
from jax.experimental.pallas import tpu as pltpu
from jax.experimental.pallas import tpu_sc as plsc

N_NODES = 10000
N_EDGES = 320000
D = 128

NC = 2
NS = 16
NW = NC * NS
CHUNK = 128
CHUNKS_PER_TILE = 80
E_PAD = NW * CHUNK * CHUNKS_PER_TILE
N_PAD = 10240
ROWS_PER_TILE = N_PAD // NS

BLK = 1024
GRID = N_PAD // BLK


def _sc_mesh():
    return plsc.VectorSubcoreMesh(core_axis_name="c", subcore_axis_name="s")


@functools.partial(
    pl.kernel,
    out_type=jax.ShapeDtypeStruct((NC, N_PAD, D), jnp.float32),
    mesh=_sc_mesh(),
    scratch_types=[
        pltpu.VMEM((CHUNKS_PER_TILE, CHUNK), jnp.int32),
        pltpu.VMEM((CHUNK, D), jnp.float32),
        pltpu.VMEM_SHARED((N_PAD, D), jnp.float32),
        pltpu.SemaphoreType.DMA,
    ],
)
def _deg_kernel(dst_hbm, ones_hbm, zero_hbm, out_hbm, dst_v, ones_v, acc, sem):
    c = lax.axis_index("c")
    s = lax.axis_index("s")
    wid = c * NS + s
    pltpu.sync_copy(dst_hbm.at[wid], dst_v)
    pltpu.sync_copy(ones_hbm, ones_v)
    pltpu.sync_copy(zero_hbm, acc.at[pl.ds(s * ROWS_PER_TILE, ROWS_PER_TILE)])
    plsc.subcore_barrier()

    k = 8

    def blk(bi, carry):
        base = bi * k
        for t in range(k):
            pltpu.async_copy(ones_v, acc.at[dst_v.at[base + t]], sem, add=True)
        for t in range(k):
            pltpu.make_async_copy(ones_v, acc.at[dst_v.at[base + t]], sem).wait()
        return carry

    lax.fori_loop(0, CHUNKS_PER_TILE // k, blk, 0)
    plsc.subcore_barrier()
    rows = pl.ds(s * ROWS_PER_TILE, ROWS_PER_TILE)
    pltpu.sync_copy(acc.at[rows], out_hbm.at[c, rows])


@functools.partial(
    pl.kernel,
    out_type=jax.ShapeDtypeStruct((NC, N_PAD, D), jnp.float32),
    mesh=_sc_mesh(),
    scratch_types=[
        pltpu.VMEM((CHUNKS_PER_TILE // 2, CHUNK), jnp.int32),
        pltpu.VMEM((CHUNKS_PER_TILE // 2, CHUNK), jnp.int32),
        pltpu.VMEM((CHUNK, D), jnp.float32),
        pltpu.VMEM((CHUNK, D), jnp.float32),
        pltpu.VMEM_SHARED((N_PAD, D), jnp.float32),
        pltpu.SemaphoreType.DMA,
        pltpu.SemaphoreType.DMA,
    ],
)
def _agg_kernel(g_hbm, src_hbm, dst_hbm, zero_hbm, out_hbm,
                src_v, dst_v, rows_a, rows_b, acc, sem_a, sem_b):
    c = lax.axis_index("c")
    s = lax.axis_index("s")
    wid = c * NS + s
    half = CHUNKS_PER_TILE // 2
    n_pairs = half // 2
    pltpu.sync_copy(zero_hbm, acc.at[pl.ds(s * ROWS_PER_TILE, ROWS_PER_TILE)])
    plsc.subcore_barrier()

    hc = CHUNK // 2

    def _fire(buf, sem, idx_row):
        pltpu.async_copy(g_hbm.at[idx_row.at[pl.ds(0, hc)]],
                         buf.at[pl.ds(0, hc)], sem)
        pltpu.async_copy(g_hbm.at[idx_row.at[pl.ds(hc, hc)]],
                         buf.at[pl.ds(hc, hc)], sem)

    def _drain(buf, sem, idx_row):
        pltpu.make_async_copy(g_hbm.at[idx_row.at[pl.ds(0, hc)]],
                              buf.at[pl.ds(0, hc)], sem).wait()
        pltpu.make_async_copy(g_hbm.at[idx_row.at[pl.ds(hc, hc)]],
                              buf.at[pl.ds(hc, hc)], sem).wait()

    for h in range(2):
        pltpu.sync_copy(src_hbm.at[wid, pl.ds(h * half, half)], src_v)
        pltpu.sync_copy(dst_hbm.at[wid, pl.ds(h * half, half)], dst_v)
        _fire(rows_a, sem_a, src_v.at[0])

        def body(jj, carry):
            j0 = 2 * jj
            _fire(rows_b, sem_b, src_v.at[j0 + 1])
            _drain(rows_a, sem_a, src_v.at[j0])
            pltpu.sync_copy(rows_a, acc.at[dst_v.at[j0]], add=True)

            @pl.when(jj + 1 < n_pairs)
            def _():
                _fire(rows_a, sem_a, src_v.at[j0 + 2])

            _drain(rows_b, sem_b, src_v.at[j0 + 1])
            pltpu.sync_copy(rows_b, acc.at[dst_v.at[j0 + 1]], add=True)
            return carry

        lax.fori_loop(0, n_pairs, body, 0)

    plsc.subcore_barrier()
    rows = pl.ds(s * ROWS_PER_TILE, ROWS_PER_TILE)
    pltpu.sync_copy(acc.at[rows], out_hbm.at[c, rows])


def _dinv(d0_ref, d1_ref):
    return lax.rsqrt(d0_ref[:, 0:1] + d1_ref[:, 0:1] + 1.0)


def _first_body(x_ref, w_ref, d0_ref, d1_ref, o_ref):
    o_ref[...] = jnp.dot(x_ref[...], w_ref[...],
                         preferred_element_type=jnp.float32) * _dinv(d0_ref, d1_ref)


def _mid_body(a0_ref, a1_ref, g_ref, d0_ref, d1_ref, w_ref, b_ref, o_ref):
    dinv = _dinv(d0_ref, d1_ref)
    z = (a0_ref[...] + a1_ref[...] + g_ref[...]) * dinv + b_ref[...]
    z = jnp.maximum(z, 0.0)
    o_ref[...] = jnp.dot(z, w_ref[...],
                         preferred_element_type=jnp.float32) * dinv


def _final_body(a0_ref, a1_ref, g_ref, d0_ref, d1_ref, b_ref, o_ref):
    dinv = _dinv(d0_ref, d1_ref)
    o_ref[...] = (a0_ref[...] + a1_ref[...] + g_ref[...]) * dinv + b_ref[...]


def _row_spec():
    return pl.BlockSpec((BLK, D), lambda i: (i, 0))


def _deg_spec():
    return pl.BlockSpec((BLK, D), lambda i: (i, 0))


def _full_spec(shape):
    return pl.BlockSpec(shape, lambda i: (0,) * len(shape))


def _tc_first(x, w, d0, d1):
    return pl.pallas_call(
        _first_body,
        grid=(GRID,),
        in_specs=[_row_spec(), _full_spec((D, D)), _deg_spec(), _deg_spec()],
        out_specs=_row_spec(),
        out_shape=jax.ShapeDtypeStruct((N_PAD, D), jnp.float32),
    )(x, w, d0, d1)


def _tc_mid(a0, a1, g, d0, d1, w, b):
    return pl.pallas_call(
        _mid_body,
        grid=(GRID,),
        in_specs=[_row_spec(), _row_spec(), _row_spec(), _deg_spec(),
                  _deg_spec(), _full_spec((D, D)), _full_spec((1, D))],
        out_specs=_row_spec(),
        out_shape=jax.ShapeDtypeStruct((N_PAD, D), jnp.float32),
    )(a0, a1, g, d0, d1, w, b)


FBLK = 1000


def _fin_spec():
    return pl.BlockSpec((FBLK, D), lambda i: (i, 0))


def _tc_final(a0, a1, g, d0, d1, b):
    return pl.pallas_call(
        _final_body,
        grid=(N_NODES // FBLK,),
        in_specs=[_fin_spec(), _fin_spec(), _fin_spec(), _fin_spec(),
                  _fin_spec(), _full_spec((1, D))],
        out_specs=_fin_spec(),
        out_shape=jax.ShapeDtypeStruct((N_NODES, D), jnp.float32),
    )(a0, a1, g, d0, d1, b)


def kernel(edge_index, input_tensor, W1, b1, W2, b2):
    src = edge_index[0].astype(jnp.int32)
    dst = edge_index[1].astype(jnp.int32)
    pad = N_NODES + (jnp.arange(E_PAD - N_EDGES, dtype=jnp.int32)
                     % (N_PAD - N_NODES))
    src3 = jnp.concatenate([src, pad]).reshape(NW, CHUNKS_PER_TILE, CHUNK)
    dst3 = jnp.concatenate([dst, pad]).reshape(NW, CHUNKS_PER_TILE, CHUNK)
    x_pad = jnp.zeros((N_PAD, D), jnp.float32).at[:N_NODES].set(input_tensor)
    ones_rows = jnp.ones((CHUNK, D), jnp.float32)
    zrows = jnp.zeros((ROWS_PER_TILE, D), jnp.float32)

    degp = _deg_kernel(dst3, ones_rows, zrows)
    d0, d1 = degp[0], degp[1]

    g1 = _tc_first(x_pad, W1, d0, d1)
    acc1 = _agg_kernel(g1, src3, dst3, zrows)
    g2 = _tc_mid(acc1[0], acc1[1], g1, d0, d1, W2, b1.reshape(1, D))
    acc2 = _agg_kernel(g2, src3, dst3, zrows)
    return _tc_final(acc2[0], acc2[1], g2, d0, d1, b2.reshape(1, D))


# speedup vs baseline: 1.0705x; 1.0705x over previous
import functools

import jax
import jax.numpy as jnp
from jax import lax
from jax.experimental import pallas as pl
from jax.experimental.pallas import tpu as pltpu
from jax.experimental.pallas import tpu_sc as plsc

N_NODES = 10000
N_EDGES = 320000
D = 128

NC = 2
NS = 16
NW = NC * NS
CHUNK = 128
CHUNKS_PER_TILE = 80
E_PAD = NW * CHUNK * CHUNKS_PER_TILE
N_PAD = 10240
ROWS_PER_TILE = N_PAD // NS

BLK = 1024
GRID = N_PAD // BLK


def _sc_mesh():
    return plsc.VectorSubcoreMesh(core_axis_name="c", subcore_axis_name="s")


DW = 64


@functools.partial(
    pl.kernel,
    out_type=jax.ShapeDtypeStruct((NC, N_PAD, DW), jnp.float32),
    mesh=_sc_mesh(),
    scratch_types=[
        pltpu.VMEM((CHUNKS_PER_TILE, CHUNK), jnp.int32),
        pltpu.VMEM((CHUNK, DW), jnp.float32),
        pltpu.VMEM_SHARED((N_PAD, DW), jnp.float32),
        pltpu.SemaphoreType.DMA,
    ],
)
def _deg_kernel(dst_hbm, ones_hbm, zero_hbm, out_hbm, dst_v, ones_v, acc, sem):
    c = lax.axis_index("c")
    s = lax.axis_index("s")
    wid = c * NS + s
    pltpu.sync_copy(dst_hbm.at[wid], dst_v)
    pltpu.sync_copy(ones_hbm, ones_v)
    pltpu.sync_copy(zero_hbm, acc.at[pl.ds(s * ROWS_PER_TILE, ROWS_PER_TILE)])
    plsc.subcore_barrier()

    k = 8

    def blk(bi, carry):
        base = bi * k
        for t in range(k):
            pltpu.async_copy(ones_v, acc.at[dst_v.at[base + t]], sem, add=True)
        for t in range(k):
            pltpu.make_async_copy(ones_v, acc.at[dst_v.at[base + t]], sem).wait()
        return carry

    lax.fori_loop(0, CHUNKS_PER_TILE // k, blk, 0)
    plsc.subcore_barrier()
    rows = pl.ds(s * ROWS_PER_TILE, ROWS_PER_TILE)
    pltpu.sync_copy(acc.at[rows], out_hbm.at[c, rows])


@functools.partial(
    pl.kernel,
    out_type=jax.ShapeDtypeStruct((NC, N_PAD, D), jnp.float32),
    mesh=_sc_mesh(),
    scratch_types=[
        pltpu.VMEM((CHUNKS_PER_TILE // 2, CHUNK), jnp.int32),
        pltpu.VMEM((CHUNKS_PER_TILE // 2, CHUNK), jnp.int32),
        pltpu.VMEM((CHUNK, D), jnp.float32),
        pltpu.VMEM((CHUNK, D), jnp.float32),
        pltpu.VMEM_SHARED((N_PAD, D), jnp.float32),
        pltpu.SemaphoreType.DMA,
        pltpu.SemaphoreType.DMA,
    ],
)
def _agg_kernel(g_hbm, src_hbm, dst_hbm, zero_hbm, out_hbm,
                src_v, dst_v, rows_a, rows_b, acc, sem_a, sem_b):
    c = lax.axis_index("c")
    s = lax.axis_index("s")
    wid = c * NS + s
    half = CHUNKS_PER_TILE // 2
    n_pairs = half // 2
    pltpu.sync_copy(zero_hbm, acc.at[pl.ds(s * ROWS_PER_TILE, ROWS_PER_TILE)])
    plsc.subcore_barrier()

    for h in range(2):
        pltpu.sync_copy(src_hbm.at[wid, pl.ds(h * half, half)], src_v)
        pltpu.sync_copy(dst_hbm.at[wid, pl.ds(h * half, half)], dst_v)
        pltpu.async_copy(g_hbm.at[src_v.at[0]], rows_a, sem_a)

        def body(jj, carry):
            j0 = 2 * jj
            pltpu.async_copy(g_hbm.at[src_v.at[j0 + 1]], rows_b, sem_b)
            pltpu.make_async_copy(g_hbm.at[src_v.at[j0]], rows_a, sem_a).wait()
            pltpu.sync_copy(rows_a, acc.at[dst_v.at[j0]], add=True)

            @pl.when(jj + 1 < n_pairs)
            def _():
                pltpu.async_copy(g_hbm.at[src_v.at[j0 + 2]], rows_a, sem_a)

            pltpu.make_async_copy(g_hbm.at[src_v.at[j0 + 1]], rows_b, sem_b).wait()
            pltpu.sync_copy(rows_b, acc.at[dst_v.at[j0 + 1]], add=True)
            return carry

        lax.fori_loop(0, n_pairs, body, 0)

    plsc.subcore_barrier()
    rows = pl.ds(s * ROWS_PER_TILE, ROWS_PER_TILE)
    pltpu.sync_copy(acc.at[rows], out_hbm.at[c, rows])


def _dinv(d0_ref, d1_ref):
    return lax.rsqrt(d0_ref[:, 0:1] + d1_ref[:, 0:1] + 1.0)


def _first_body(x_ref, w_ref, d0_ref, d1_ref, o_ref):
    o_ref[...] = jnp.dot(x_ref[...], w_ref[...],
                         preferred_element_type=jnp.float32) * _dinv(d0_ref, d1_ref)


def _mid_body(a0_ref, a1_ref, g_ref, d0_ref, d1_ref, w_ref, b_ref, o_ref):
    dinv = _dinv(d0_ref, d1_ref)
    z = (a0_ref[...] + a1_ref[...] + g_ref[...]) * dinv + b_ref[...]
    z = jnp.maximum(z, 0.0)
    o_ref[...] = jnp.dot(z, w_ref[...],
                         preferred_element_type=jnp.float32) * dinv


def _final_body(a0_ref, a1_ref, g_ref, d0_ref, d1_ref, b_ref, o_ref):
    dinv = _dinv(d0_ref, d1_ref)
    o_ref[...] = (a0_ref[...] + a1_ref[...] + g_ref[...]) * dinv + b_ref[...]


def _row_spec():
    return pl.BlockSpec((BLK, D), lambda i: (i, 0))


def _deg_spec():
    return pl.BlockSpec((BLK, DW), lambda i: (i, 0))


def _full_spec(shape):
    return pl.BlockSpec(shape, lambda i: (0,) * len(shape))


def _tc_first(x, w, d0, d1):
    return pl.pallas_call(
        _first_body,
        grid=(GRID,),
        in_specs=[_row_spec(), _full_spec((D, D)), _deg_spec(), _deg_spec()],
        out_specs=_row_spec(),
        out_shape=jax.ShapeDtypeStruct((N_PAD, D), jnp.float32),
    )(x, w, d0, d1)


def _tc_mid(a0, a1, g, d0, d1, w, b):
    return pl.pallas_call(
        _mid_body,
        grid=(GRID,),
        in_specs=[_row_spec(), _row_spec(), _row_spec(), _deg_spec(),
                  _deg_spec(), _full_spec((D, D)), _full_spec((1, D))],
        out_specs=_row_spec(),
        out_shape=jax.ShapeDtypeStruct((N_PAD, D), jnp.float32),
    )(a0, a1, g, d0, d1, w, b)


FBLK = 1000


def _fin_spec():
    return pl.BlockSpec((FBLK, D), lambda i: (i, 0))


def _tc_final(a0, a1, g, d0, d1, b):
    return pl.pallas_call(
        _final_body,
        grid=(N_NODES // FBLK,),
        in_specs=[_fin_spec(), _fin_spec(), _fin_spec(),
                  pl.BlockSpec((FBLK, DW), lambda i: (i, 0)),
                  pl.BlockSpec((FBLK, DW), lambda i: (i, 0)),
                  _full_spec((1, D))],
        out_specs=_fin_spec(),
        out_shape=jax.ShapeDtypeStruct((N_NODES, D), jnp.float32),
    )(a0, a1, g, d0, d1, b)


def kernel(edge_index, input_tensor, W1, b1, W2, b2):
    src = edge_index[0].astype(jnp.int32)
    dst = edge_index[1].astype(jnp.int32)
    pad = N_NODES + (jnp.arange(E_PAD - N_EDGES, dtype=jnp.int32)
                     % (N_PAD - N_NODES))
    src3 = jnp.concatenate([src, pad]).reshape(NW, CHUNKS_PER_TILE, CHUNK)
    dst3 = jnp.concatenate([dst, pad]).reshape(NW, CHUNKS_PER_TILE, CHUNK)
    x_pad = jnp.zeros((N_PAD, D), jnp.float32).at[:N_NODES].set(input_tensor)
    ones_rows = jnp.ones((CHUNK, DW), jnp.float32)
    zdeg = jnp.zeros((ROWS_PER_TILE, DW), jnp.float32)
    zrows = jnp.zeros((ROWS_PER_TILE, D), jnp.float32)

    degp = _deg_kernel(dst3, ones_rows, zdeg)
    d0, d1 = degp[0], degp[1]

    g1 = _tc_first(x_pad, W1, d0, d1)
    acc1 = _agg_kernel(g1, src3, dst3, zrows)
    g2 = _tc_mid(acc1[0], acc1[1], g1, d0, d1, W2, b1.reshape(1, D))
    acc2 = _agg_kernel(g2, src3, dst3, zrows)
    return _tc_final(acc2[0], acc2[1], g2, d0, d1, b2.reshape(1, D))


# speedup vs baseline: 1.1123x; 1.0390x over previous
import functools

import jax
import jax.numpy as jnp
from jax import lax
from jax.experimental import pallas as pl
from jax.experimental.pallas import tpu as pltpu
from jax.experimental.pallas import tpu_sc as plsc

N_NODES = 10000
N_EDGES = 320000
D = 128

NC = 2
NS = 16
NW = NC * NS
CHUNK = 128
CHUNKS_PER_TILE = 80
E_PAD = NW * CHUNK * CHUNKS_PER_TILE
N_PAD = 10240
ROWS_PER_TILE = N_PAD // NS

BLK = 1024
GRID = N_PAD // BLK


def _sc_mesh():
    return plsc.VectorSubcoreMesh(core_axis_name="c", subcore_axis_name="s")


DW = 32


@functools.partial(
    pl.kernel,
    out_type=jax.ShapeDtypeStruct((NC, N_PAD, DW), jnp.float32),
    mesh=_sc_mesh(),
    scratch_types=[
        pltpu.VMEM((CHUNKS_PER_TILE, CHUNK), jnp.int32),
        pltpu.VMEM((CHUNK, DW), jnp.float32),
        pltpu.VMEM_SHARED((N_PAD, DW), jnp.float32),
        pltpu.SemaphoreType.DMA,
    ],
)
def _deg_kernel(dst_hbm, ones_hbm, zero_hbm, out_hbm, dst_v, ones_v, acc, sem):
    c = lax.axis_index("c")
    s = lax.axis_index("s")
    wid = c * NS + s
    pltpu.sync_copy(dst_hbm.at[wid], dst_v)
    pltpu.sync_copy(ones_hbm, ones_v)
    pltpu.sync_copy(zero_hbm, acc.at[pl.ds(s * ROWS_PER_TILE, ROWS_PER_TILE)])
    plsc.subcore_barrier()

    k = 8

    def blk(bi, carry):
        base = bi * k
        for t in range(k):
            pltpu.async_copy(ones_v, acc.at[dst_v.at[base + t]], sem, add=True)
        for t in range(k):
            pltpu.make_async_copy(ones_v, acc.at[dst_v.at[base + t]], sem).wait()
        return carry

    lax.fori_loop(0, CHUNKS_PER_TILE // k, blk, 0)
    plsc.subcore_barrier()
    rows = pl.ds(s * ROWS_PER_TILE, ROWS_PER_TILE)
    pltpu.sync_copy(acc.at[rows], out_hbm.at[c, rows])


@functools.partial(
    pl.kernel,
    out_type=jax.ShapeDtypeStruct((NC, N_PAD, D), jnp.float32),
    mesh=_sc_mesh(),
    scratch_types=[
        pltpu.VMEM((CHUNKS_PER_TILE // 2, CHUNK), jnp.int32),
        pltpu.VMEM((CHUNKS_PER_TILE // 2, CHUNK), jnp.int32),
        pltpu.VMEM((CHUNK, D), jnp.float32),
        pltpu.VMEM((CHUNK, D), jnp.float32),
        pltpu.VMEM_SHARED((N_PAD, D), jnp.float32),
        pltpu.SemaphoreType.DMA,
        pltpu.SemaphoreType.DMA,
    ],
)
def _agg_kernel(g_hbm, src_hbm, dst_hbm, zero_hbm, out_hbm,
                src_v, dst_v, rows_a, rows_b, acc, sem_a, sem_b):
    c = lax.axis_index("c")
    s = lax.axis_index("s")
    wid = c * NS + s
    half = CHUNKS_PER_TILE // 2
    n_pairs = half // 2
    pltpu.sync_copy(zero_hbm, acc.at[pl.ds(s * ROWS_PER_TILE, ROWS_PER_TILE)])
    plsc.subcore_barrier()

    for h in range(2):
        pltpu.sync_copy(src_hbm.at[wid, pl.ds(h * half, half)], src_v)
        pltpu.sync_copy(dst_hbm.at[wid, pl.ds(h * half, half)], dst_v)
        pltpu.async_copy(g_hbm.at[src_v.at[0]], rows_a, sem_a)

        def body(jj, carry):
            j0 = 2 * jj
            pltpu.async_copy(g_hbm.at[src_v.at[j0 + 1]], rows_b, sem_b)
            pltpu.make_async_copy(g_hbm.at[src_v.at[j0]], rows_a, sem_a).wait()
            pltpu.sync_copy(rows_a, acc.at[dst_v.at[j0]], add=True)

            @pl.when(jj + 1 < n_pairs)
            def _():
                pltpu.async_copy(g_hbm.at[src_v.at[j0 + 2]], rows_a, sem_a)

            pltpu.make_async_copy(g_hbm.at[src_v.at[j0 + 1]], rows_b, sem_b).wait()
            pltpu.sync_copy(rows_b, acc.at[dst_v.at[j0 + 1]], add=True)
            return carry

        lax.fori_loop(0, n_pairs, body, 0)

    plsc.subcore_barrier()
    rows = pl.ds(s * ROWS_PER_TILE, ROWS_PER_TILE)
    pltpu.sync_copy(acc.at[rows], out_hbm.at[c, rows])


def _dinv(d0_ref, d1_ref):
    return lax.rsqrt(d0_ref[:, 0:1] + d1_ref[:, 0:1] + 1.0)


def _first_body(x_ref, w_ref, d0_ref, d1_ref, o_ref):
    o_ref[...] = jnp.dot(x_ref[...], w_ref[...],
                         preferred_element_type=jnp.float32) * _dinv(d0_ref, d1_ref)


def _mid_body(a0_ref, a1_ref, g_ref, d0_ref, d1_ref, w_ref, b_ref, o_ref):
    dinv = _dinv(d0_ref, d1_ref)
    z = (a0_ref[...] + a1_ref[...] + g_ref[...]) * dinv + b_ref[...]
    z = jnp.maximum(z, 0.0)
    o_ref[...] = jnp.dot(z, w_ref[...],
                         preferred_element_type=jnp.float32) * dinv


def _final_body(a0_ref, a1_ref, g_ref, d0_ref, d1_ref, b_ref, o_ref):
    dinv = _dinv(d0_ref, d1_ref)
    o_ref[...] = (a0_ref[...] + a1_ref[...] + g_ref[...]) * dinv + b_ref[...]


def _row_spec():
    return pl.BlockSpec((BLK, D), lambda i: (i, 0))


def _deg_spec():
    return pl.BlockSpec((BLK, DW), lambda i: (i, 0))


def _full_spec(shape):
    return pl.BlockSpec(shape, lambda i: (0,) * len(shape))


def _tc_first(x, w, d0, d1):
    return pl.pallas_call(
        _first_body,
        grid=(GRID,),
        in_specs=[_row_spec(), _full_spec((D, D)), _deg_spec(), _deg_spec()],
        out_specs=_row_spec(),
        out_shape=jax.ShapeDtypeStruct((N_PAD, D), jnp.float32),
    )(x, w, d0, d1)


def _tc_mid(a0, a1, g, d0, d1, w, b):
    return pl.pallas_call(
        _mid_body,
        grid=(GRID,),
        in_specs=[_row_spec(), _row_spec(), _row_spec(), _deg_spec(),
                  _deg_spec(), _full_spec((D, D)), _full_spec((1, D))],
        out_specs=_row_spec(),
        out_shape=jax.ShapeDtypeStruct((N_PAD, D), jnp.float32),
    )(a0, a1, g, d0, d1, w, b)


FBLK = 1000


def _fin_spec():
    return pl.BlockSpec((FBLK, D), lambda i: (i, 0))


def _tc_final(a0, a1, g, d0, d1, b):
    return pl.pallas_call(
        _final_body,
        grid=(N_NODES // FBLK,),
        in_specs=[_fin_spec(), _fin_spec(), _fin_spec(),
                  pl.BlockSpec((FBLK, DW), lambda i: (i, 0)),
                  pl.BlockSpec((FBLK, DW), lambda i: (i, 0)),
                  _full_spec((1, D))],
        out_specs=_fin_spec(),
        out_shape=jax.ShapeDtypeStruct((N_NODES, D), jnp.float32),
    )(a0, a1, g, d0, d1, b)


def kernel(edge_index, input_tensor, W1, b1, W2, b2):
    src = edge_index[0].astype(jnp.int32)
    dst = edge_index[1].astype(jnp.int32)
    pad = N_NODES + (jnp.arange(E_PAD - N_EDGES, dtype=jnp.int32)
                     % (N_PAD - N_NODES))
    src3 = jnp.concatenate([src, pad]).reshape(NW, CHUNKS_PER_TILE, CHUNK)
    dst3 = jnp.concatenate([dst, pad]).reshape(NW, CHUNKS_PER_TILE, CHUNK)
    x_pad = jnp.zeros((N_PAD, D), jnp.float32).at[:N_NODES].set(input_tensor)
    ones_rows = jnp.ones((CHUNK, DW), jnp.float32)
    zdeg = jnp.zeros((ROWS_PER_TILE, DW), jnp.float32)
    zrows = jnp.zeros((ROWS_PER_TILE, D), jnp.float32)

    degp = _deg_kernel(dst3, ones_rows, zdeg)
    d0, d1 = degp[0], degp[1]

    g1 = _tc_first(x_pad, W1, d0, d1)
    acc1 = _agg_kernel(g1, src3, dst3, zrows)
    g2 = _tc_mid(acc1[0], acc1[1], g1, d0, d1, W2, b1.reshape(1, D))
    acc2 = _agg_kernel(g2, src3, dst3, zrows)
    return _tc_final(acc2[0], acc2[1], g2, d0, d1, b2.reshape(1, D))
